# baseline (speedup 1.0000x reference)
import dataclasses
import functools

import jax
import jax.numpy as jnp
from jax import lax
from jax.experimental import pallas as pl
from jax.experimental.pallas import tpu as pltpu
from jax.experimental.pallas import tpu_sc as plsc

ROWS = 64
COLS = 32768
NUM_CORES = 2
NUM_SUBCORES = 16
LANES = 16
NUM_WORKERS = NUM_CORES * NUM_SUBCORES
SC_ROWS = NUM_WORKERS
TC_ROWS = ROWS - SC_ROWS
INT_MAX = 2**31 - 1

NACC = 8
PIECES = (2048, 2048, 4096, 8192, 16384)
assert sum(PIECES) == COLS


def _compiler_params():
    cp = pltpu.CompilerParams()
    if "needs_layout_passes" in pltpu.CompilerParams.__dataclass_fields__:
        cp = dataclasses.replace(cp, needs_layout_passes=False)
    return cp


def _scan_piece(buf, base_chunk, n_chunks, carry):
    def body(i, c):
        vals, chunks = c
        new_vals, new_chunks = [], []
        for j in range(NACC):
            ch = base_chunk + i * NACC + j
            v = buf[pl.ds(ch * LANES, LANES)]
            m = v > vals[j]
            new_vals.append(jnp.where(m, v, vals[j]))
            new_chunks.append(
                jnp.where(m, jnp.full((LANES,), ch, jnp.int32), chunks[j]))
        return tuple(new_vals), tuple(new_chunks)

    return lax.fori_loop(0, n_chunks // NACC, body, carry, unroll=4)


def _finish_row(carry):
    vals, chunks = carry
    best_val, best_chunk = vals[0], chunks[0]
    for j in range(1, NACC):
        take = (vals[j] > best_val) | ((vals[j] == best_val)
                                       & (chunks[j] < best_chunk))
        best_val = jnp.where(take, vals[j], best_val)
        best_chunk = jnp.where(take, chunks[j], best_chunk)
    lane = lax.iota(jnp.int32, LANES)
    idx = best_chunk * LANES + lane
    row_max = jnp.max(best_val)
    cand = jnp.where(best_val == row_max, idx,
                     jnp.full((LANES,), INT_MAX, jnp.int32))
    return jnp.min(cand)


def _fresh_carry():
    neg_inf = jnp.float32(float("-inf"))
    return (
        tuple(jnp.full((LANES,), neg_inf, jnp.float32) for _ in range(NACC)),
        tuple(jnp.zeros((LANES,), jnp.int32) for _ in range(NACC)),
    )


def _sc_argmax(x):
    mesh = plsc.VectorSubcoreMesh(core_axis_name="c", subcore_axis_name="s")

    @functools.partial(
        pl.kernel,
        out_type=jax.ShapeDtypeStruct((SC_ROWS, LANES), jnp.int32),
        mesh=mesh,
        compiler_params=_compiler_params(),
        scratch_types=[
            pltpu.VMEM((COLS,), jnp.float32),
            pltpu.VMEM((LANES,), jnp.int32),
        ] + [pltpu.SemaphoreType.DMA] * len(PIECES),
    )
    def argmax_kernel(x_hbm, out_hbm, row_v, res_v, *sems):
        wid = lax.axis_index("c") * NUM_SUBCORES + lax.axis_index("s")

        copies = []
        off = 0
        for p, sz in enumerate(PIECES):
            copies.append(pltpu.async_copy(
                x_hbm.at[wid, pl.ds(off, sz)],
                row_v.at[pl.ds(off, sz)], sems[p]))
            off += sz

        carry = _fresh_carry()
        off = 0
        for p, sz in enumerate(PIECES):
            copies[p].wait()
            carry = _scan_piece(row_v, off // LANES, sz // LANES, carry)
            off += sz
        r = _finish_row(carry)

        lane = lax.iota(jnp.int32, LANES)
        res_v[...] = jnp.where(lane == 0, jnp.full((LANES,), r, jnp.int32),
                               jnp.zeros((LANES,), jnp.int32))
        pltpu.sync_copy(res_v, out_hbm.at[wid])

    return argmax_kernel(x)


def _tc_argmax_kernel(x_ref, out_ref):
    x = x_ref[...]
    row_max = jnp.max(x, axis=1, keepdims=True)
    ii = lax.broadcasted_iota(jnp.int32, x.shape, 1)
    cand = jnp.where(x == row_max, ii, INT_MAX)
    out_ref[...] = jnp.min(cand, axis=1)


def _tc_argmax(x):
    return pl.pallas_call(
        _tc_argmax_kernel,
        grid=(1,),
        in_specs=[pl.BlockSpec((TC_ROWS, COLS), lambda i: (1, 0))],
        out_specs=pl.BlockSpec((TC_ROWS,), lambda i: (0,)),
        out_shape=jax.ShapeDtypeStruct((TC_ROWS,), jnp.int32),
    )(x)


def _combine_kernel(sc_ref, tc_ref, out_ref):
    col = lax.broadcasted_iota(jnp.int32, (SC_ROWS, LANES), 1)
    sc = jnp.sum(jnp.where(col == 0, sc_ref[...], 0), axis=1)
    out_ref[pl.ds(0, SC_ROWS)] = sc
    out_ref[pl.ds(SC_ROWS, TC_ROWS)] = tc_ref[...]


def _combine(sc_out, tc_out):
    return pl.pallas_call(
        _combine_kernel,
        out_shape=jax.ShapeDtypeStruct((ROWS,), jnp.int32),
    )(sc_out, tc_out)


def kernel(x):
    sc_out = _sc_argmax(x)
    tc_out = _tc_argmax(x)
    return _combine(sc_out, tc_out)


# speedup vs baseline: 1.0749x; 1.0749x over previous
import dataclasses
import functools

import jax
import jax.numpy as jnp
from jax import lax
from jax.experimental import pallas as pl
from jax.experimental.pallas import tpu as pltpu
from jax.experimental.pallas import tpu_sc as plsc

ROWS = 64
COLS = 32768
NUM_CORES = 2
NUM_SUBCORES = 16
LANES = 16
NUM_WORKERS = NUM_CORES * NUM_SUBCORES
SC_ROWS = NUM_WORKERS
TC_ROWS = ROWS - SC_ROWS
INT_MAX = 2**31 - 1

NACC = 8
PIECES = (2048, 2048, 4096, 8192, 16384)
assert sum(PIECES) == COLS


def _compiler_params():
    cp = pltpu.CompilerParams()
    if "needs_layout_passes" in pltpu.CompilerParams.__dataclass_fields__:
        cp = dataclasses.replace(cp, needs_layout_passes=False)
    return cp


def _scan_piece(buf, base_chunk, n_chunks, carry):
    def body(i, c):
        vals, chunks = c
        new_vals, new_chunks = [], []
        for j in range(NACC):
            ch = base_chunk + i * NACC + j
            v = buf[pl.ds(ch * LANES, LANES)]
            m = v > vals[j]
            new_vals.append(jnp.where(m, v, vals[j]))
            new_chunks.append(
                jnp.where(m, jnp.full((LANES,), ch, jnp.int32), chunks[j]))
        return tuple(new_vals), tuple(new_chunks)

    return lax.fori_loop(0, n_chunks // NACC, body, carry, unroll=1)


def _finish_row(carry):
    vals, chunks = carry
    best_val, best_chunk = vals[0], chunks[0]
    for j in range(1, NACC):
        take = (vals[j] > best_val) | ((vals[j] == best_val)
                                       & (chunks[j] < best_chunk))
        best_val = jnp.where(take, vals[j], best_val)
        best_chunk = jnp.where(take, chunks[j], best_chunk)
    lane = lax.iota(jnp.int32, LANES)
    idx = best_chunk * LANES + lane
    row_max = jnp.max(best_val)
    cand = jnp.where(best_val == row_max, idx,
                     jnp.full((LANES,), INT_MAX, jnp.int32))
    return jnp.min(cand)


def _fresh_carry():
    neg_inf = jnp.float32(float("-inf"))
    return (
        tuple(jnp.full((LANES,), neg_inf, jnp.float32) for _ in range(NACC)),
        tuple(jnp.zeros((LANES,), jnp.int32) for _ in range(NACC)),
    )


def _sc_argmax(x):
    mesh = plsc.VectorSubcoreMesh(core_axis_name="c", subcore_axis_name="s")

    @functools.partial(
        pl.kernel,
        out_type=jax.ShapeDtypeStruct((SC_ROWS, LANES), jnp.int32),
        mesh=mesh,
        compiler_params=_compiler_params(),
        scratch_types=[
            pltpu.VMEM((COLS,), jnp.float32),
            pltpu.VMEM((LANES,), jnp.int32),
        ] + [pltpu.SemaphoreType.DMA] * len(PIECES),
    )
    def argmax_kernel(x_hbm, out_hbm, row_v, res_v, *sems):
        wid = lax.axis_index("c") * NUM_SUBCORES + lax.axis_index("s")

        copies = []
        off = 0
        for p, sz in enumerate(PIECES):
            copies.append(pltpu.async_copy(
                x_hbm.at[wid, pl.ds(off, sz)],
                row_v.at[pl.ds(off, sz)], sems[p]))
            off += sz

        carry = _fresh_carry()
        off = 0
        for p, sz in enumerate(PIECES):
            copies[p].wait()
            carry = _scan_piece(row_v, off // LANES, sz // LANES, carry)
            off += sz
        r = _finish_row(carry)

        lane = lax.iota(jnp.int32, LANES)
        res_v[...] = jnp.where(lane == 0, jnp.full((LANES,), r, jnp.int32),
                               jnp.zeros((LANES,), jnp.int32))
        pltpu.sync_copy(res_v, out_hbm.at[wid])

    return argmax_kernel(x)


def _tc_argmax_kernel(x_ref, out_ref):
    x = x_ref[...]
    row_max = jnp.max(x, axis=1, keepdims=True)
    ii = lax.broadcasted_iota(jnp.int32, x.shape, 1)
    cand = jnp.where(x == row_max, ii, INT_MAX)
    out_ref[...] = jnp.min(cand, axis=1)


def _tc_argmax(x):
    return pl.pallas_call(
        _tc_argmax_kernel,
        grid=(1,),
        in_specs=[pl.BlockSpec((TC_ROWS, COLS), lambda i: (1, 0))],
        out_specs=pl.BlockSpec((TC_ROWS,), lambda i: (0,)),
        out_shape=jax.ShapeDtypeStruct((TC_ROWS,), jnp.int32),
    )(x)


def _combine_kernel(sc_ref, tc_ref, out_ref):
    col = lax.broadcasted_iota(jnp.int32, (SC_ROWS, LANES), 1)
    sc = jnp.sum(jnp.where(col == 0, sc_ref[...], 0), axis=1)
    out_ref[pl.ds(0, SC_ROWS)] = sc
    out_ref[pl.ds(SC_ROWS, TC_ROWS)] = tc_ref[...]


def _combine(sc_out, tc_out):
    return pl.pallas_call(
        _combine_kernel,
        out_shape=jax.ShapeDtypeStruct((ROWS,), jnp.int32),
    )(sc_out, tc_out)


def kernel(x):
    sc_out = _sc_argmax(x)
    tc_out = _tc_argmax(x)
    return _combine(sc_out, tc_out)
